# Initial kernel scaffold; baseline (speedup 1.0000x reference)
import jax
import jax.numpy as jnp
from jax.experimental import pallas as pl


def kernel(x, eu_gmf, ei_gmf, eu_mlp, ei_mlp, W0, b0, W1, b1, W2, b2, Wp, bp):
    raise NotImplementedError("write your pallas kernel here")


# speedup vs baseline: 1.1458x; 1.1458x over previous
import functools

import jax
import jax.numpy as jnp
from jax.experimental import pallas as pl
from jax.experimental.pallas import tpu as pltpu
from jax.experimental.pallas import tpu_sc as plsc

B = 16384
GMF_D = 32
MLP_D = 128
_W = 128

@functools.cache
def _sc_gather_fn():
    mesh = plsc.VectorSubcoreMesh(core_axis_name="core",
                                  subcore_axis_name="subcore")

    @functools.partial(
        pl.kernel,
        out_type=(
            jax.ShapeDtypeStruct((B, GMF_D), jnp.float32),
            jax.ShapeDtypeStruct((B, GMF_D), jnp.float32),
            jax.ShapeDtypeStruct((B, MLP_D), jnp.float32),
            jax.ShapeDtypeStruct((B, MLP_D), jnp.float32),
        ),
        mesh=mesh,
        compiler_params=pltpu.CompilerParams(use_tc_tiling_on_sc=False),
    )
    def _sc_gather(uidx_hbm, iidx_hbm, eu_gmf_hbm, ei_gmf_hbm, eu_mlp_hbm,
                   ei_mlp_hbm, gu_hbm, gi_hbm, mu_hbm, mi_hbm):
        def body(u_v, i_v, gu_v, gi_v, mu_v, mi_v):
            pltpu.sync_copy(eu_gmf_hbm.at[u_v.at[0]], gu_v)
            pltpu.sync_copy(ei_gmf_hbm.at[i_v.at[0]], gi_v)
            pltpu.sync_copy(eu_mlp_hbm.at[u_v.at[0]], mu_v)
            pltpu.sync_copy(ei_mlp_hbm.at[i_v.at[0]], mi_v)

        pltpu.emit_pipeline(
            body,
            grid=(B // _W,),
            in_specs=[
                pl.BlockSpec((1, _W), lambda i: (0, i)),
                pl.BlockSpec((1, _W), lambda i: (0, i)),
            ],
            out_specs=[
                pl.BlockSpec((_W, GMF_D), lambda i: (i, 0)),
                pl.BlockSpec((_W, GMF_D), lambda i: (i, 0)),
                pl.BlockSpec((_W, MLP_D), lambda i: (i, 0)),
                pl.BlockSpec((_W, MLP_D), lambda i: (i, 0)),
            ],
            core_axis_name=("core", "subcore"),
            dimension_semantics=(pltpu.PARALLEL,),
        )(uidx_hbm, iidx_hbm, gu_hbm, gi_hbm, mu_hbm, mi_hbm)

    return _sc_gather


_BLK = 2048


def _tc_body(gu, gi, mu, mi, w0u, w0i, b0, w1, b1, w2, b2, wpg, wpm, bp, out):
    h = jnp.dot(mu[...], w0u[...], preferred_element_type=jnp.float32)
    h = h + jnp.dot(mi[...], w0i[...], preferred_element_type=jnp.float32)
    h = jnp.maximum(h + b0[...], 0.0)
    h = jnp.maximum(
        jnp.dot(h, w1[...], preferred_element_type=jnp.float32) + b1[...], 0.0)
    h = jnp.maximum(
        jnp.dot(h, w2[...], preferred_element_type=jnp.float32) + b2[...], 0.0)
    g = gu[...] * gi[...]
    pred = (jnp.sum(g * wpg[...], axis=1) + jnp.sum(h * wpm[...], axis=1)
            + bp[0, 0])
    out[...] = pred


def _tc_fuse(gu, gi, mu, mi, w0u, w0i, b0, w1, b1, w2, b2, wpg, wpm, bp):
    n_blk = B // _BLK
    batch_spec = lambda d: pl.BlockSpec((_BLK, d), lambda i: (i, 0))
    full = lambda a: pl.BlockSpec(a.shape, lambda i: (0,) * a.ndim)
    return pl.pallas_call(
        _tc_body,
        grid=(n_blk,),
        in_specs=[
            batch_spec(GMF_D), batch_spec(GMF_D),
            batch_spec(MLP_D), batch_spec(MLP_D),
            full(w0u), full(w0i), full(b0), full(w1), full(b1),
            full(w2), full(b2), full(wpg), full(wpm), full(bp),
        ],
        out_specs=pl.BlockSpec((_BLK,), lambda i: (i,)),
        out_shape=jax.ShapeDtypeStruct((B,), jnp.float32),
    )(gu, gi, mu, mi, w0u, w0i, b0, w1, b1, w2, b2, wpg, wpm, bp)


def kernel(x, eu_gmf, ei_gmf, eu_mlp, ei_mlp, W0, b0, W1, b1, W2, b2, Wp, bp):
    uidx = x[:, 0].reshape(1, B)
    iidx = x[:, 1].reshape(1, B)
    gu, gi, mu, mi = _sc_gather_fn()(uidx, iidx, eu_gmf, ei_gmf, eu_mlp,
                                     ei_mlp)
    pred = _tc_fuse(
        gu, gi, mu, mi,
        W0[:, :MLP_D].T, W0[:, MLP_D:].T, b0.reshape(1, -1),
        W1.T, b1.reshape(1, -1), W2.T, b2.reshape(1, -1),
        Wp[:, :GMF_D], Wp[:, GMF_D:], bp.reshape(1, 1),
    )
    return pred.reshape(-1)
